# Initial kernel scaffold; baseline (speedup 1.0000x reference)
import jax
import jax.numpy as jnp
from jax.experimental import pallas as pl


def kernel(x, edge_index, W1, b1, W2, b2):
    raise NotImplementedError("write your pallas kernel here")


# speedup vs baseline: 13.2201x; 13.2201x over previous
import functools

import jax
import jax.numpy as jnp
from jax import lax
from jax.experimental import pallas as pl
from jax.experimental.pallas import tpu as pltpu
from jax.experimental.pallas import tpu_sc as plsc

NC = 2
NS = 16
LN = 16


def _cdiv(a, b):
    return (a + b - 1) // b


def _sc_mesh():
    return plsc.VectorSubcoreMesh(
        core_axis_name="c", subcore_axis_name="s", num_cores=NC, num_subcores=NS
    )


def _make_deg_kernel(NPAD, EPAD):
    EPT = EPAD // (NC * NS)
    CH = EPT // 128
    SLAB = NPAD // NS

    @functools.partial(
        pl.kernel,
        out_type=jax.ShapeDtypeStruct((NC, NPAD), jnp.float32),
        mesh=_sc_mesh(),
        scratch_types=[
            pltpu.VMEM((CH, 128), jnp.int32),
            pltpu.VMEM((128,), jnp.float32),
            pltpu.VMEM((SLAB,), jnp.float32),
            pltpu.VMEM_SHARED((NPAD,), jnp.float32),
        ],
    )
    def deg_kernel(dst_hbm, out_hbm, idx_v, ones_v, zbuf, sdeg):
        c = lax.axis_index("c")
        s = lax.axis_index("s")
        wid = s * NC + c

        def zero(i, _):
            zbuf[pl.ds(i * LN, LN)] = jnp.zeros((LN,), jnp.float32)
            return 0

        lax.fori_loop(0, SLAB // LN, zero, 0)

        def one(i, _):
            ones_v[pl.ds(i * LN, LN)] = jnp.full((LN,), 1.0, jnp.float32)
            return 0

        lax.fori_loop(0, 128 // LN, one, 0)
        pltpu.sync_copy(zbuf, sdeg.at[pl.ds(s * SLAB, SLAB)])
        pltpu.sync_copy(dst_hbm.at[pl.ds(wid * CH, CH)], idx_v)
        plsc.subcore_barrier()

        def body(j, _):
            pltpu.sync_copy(ones_v, sdeg.at[idx_v.at[j]], add=True)
            return 0

        lax.fori_loop(0, CH, body, 0)
        plsc.subcore_barrier()
        pltpu.sync_copy(sdeg.at[pl.ds(s * SLAB, SLAB)], out_hbm.at[c, pl.ds(s * SLAB, SLAB)])

    return deg_kernel


def _make_sc_edge_kernel(NPAD, EPAD, D):
    EPT = EPAD // NS
    CH = EPT // 128
    SC_CH = 16
    NSUP = CH // SC_CH
    SLAB = NPAD // NS

    @functools.partial(
        pl.kernel,
        out_type=jax.ShapeDtypeStruct((NC, NPAD, D), jnp.float32),
        mesh=_sc_mesh(),
        scratch_types=[
            pltpu.VMEM((SC_CH, 128), jnp.int32),
            pltpu.VMEM((SC_CH, 128), jnp.int32),
            pltpu.VMEM((128, D), jnp.float32),
            pltpu.VMEM((128, D), jnp.float32),
            pltpu.VMEM_SHARED((NPAD, D), jnp.float32),
            pltpu.SemaphoreType.DMA,
            pltpu.SemaphoreType.DMA,
        ],
        compiler_params=pltpu.CompilerParams(use_tc_tiling_on_sc=False),
    )
    def edge_kernel(h_hbm, src_hbm, dst_hbm, out_hbm,
                    sidx, didx, ga, gb, accum, sa, sb):
        c = lax.axis_index("c")
        s = lax.axis_index("s")
        r0 = s * SLAB
        pltpu.sync_copy(h_hbm.at[c, pl.ds(r0, SLAB)], accum.at[pl.ds(r0, SLAB)])
        plsc.subcore_barrier()
        table = h_hbm.at[c]

        def body(g, _):
            base = s * CH + g * SC_CH
            pltpu.sync_copy(src_hbm.at[pl.ds(base, SC_CH)], sidx)
            pltpu.sync_copy(dst_hbm.at[pl.ds(base, SC_CH)], didx)
            pltpu.async_copy(table.at[sidx.at[0]], ga, sa)
            for j in range(SC_CH):
                buf, sem = (ga, sa) if j % 2 == 0 else (gb, sb)
                nbuf, nsem = (gb, sb) if j % 2 == 0 else (ga, sa)
                pltpu.make_async_copy(table.at[sidx.at[j]], buf, sem).wait()
                if j + 1 < SC_CH:
                    pltpu.async_copy(table.at[sidx.at[j + 1]], nbuf, nsem)
                pltpu.sync_copy(buf, accum.at[didx.at[j]], add=True)
            return 0

        lax.fori_loop(0, NSUP, body, 0)
        plsc.subcore_barrier()
        pltpu.sync_copy(accum.at[pl.ds(r0, SLAB)], out_hbm.at[c, pl.ds(r0, SLAB)])

    return edge_kernel


def _tc_pre(NPAD, N, Dhid):
    H = Dhid // 2

    def body(dp_ref, x_ref, w_ref, d_ref, hs_ref):
        deg = jnp.sum(dp_ref[...], axis=0) + 1.0
        d = lax.rsqrt(deg)
        dcol = d.reshape(NPAD, 1)
        d_ref[...] = dcol
        h = jnp.dot(x_ref[...], w_ref[...], preferred_element_type=jnp.float32)
        hs = h * dcol[:N]
        hs_ref[0, :N, :] = hs[:, :H]
        hs_ref[1, :N, :] = hs[:, H:]

    return pl.pallas_call(
        body,
        out_shape=[
            jax.ShapeDtypeStruct((NPAD, 1), jnp.float32),
            jax.ShapeDtypeStruct((NC, NPAD, H), jnp.float32),
        ],
    )


def _tc_mid(NPAD, N, Dhid, Dout):
    H = Dout // 2

    def body(a_ref, d_ref, b1_ref, w2_ref, hs_ref):
        acc = jnp.concatenate([a_ref[0, :N, :], a_ref[1, :N, :]], axis=1)
        dcol = d_ref[:N]
        h = jnp.maximum(acc * dcol + b1_ref[...], 0.0)
        h2 = jnp.dot(h, w2_ref[...], preferred_element_type=jnp.float32)
        hs = h2 * dcol
        hs_ref[0, :N, :] = hs[:, :H]
        hs_ref[1, :N, :] = hs[:, H:]

    return pl.pallas_call(
        body,
        out_shape=[jax.ShapeDtypeStruct((NC, NPAD, H), jnp.float32)],
    )


def _tc_post(NPAD, N, Dout):
    def body(a_ref, d_ref, b2_ref, out_ref):
        acc = jnp.concatenate([a_ref[0, :N, :], a_ref[1, :N, :]], axis=1)
        out_ref[...] = acc * d_ref[:N] + b2_ref[...]

    return pl.pallas_call(
        body,
        out_shape=jax.ShapeDtypeStruct((N, Dout), jnp.float32),
    )


def kernel(x, edge_index, W1, b1, W2, b2):
    N, Din = x.shape
    E = edge_index.shape[1]
    Dhid = W1.shape[1]
    Dout = W2.shape[1]

    NPAD = _cdiv(N + 1, 256) * 256
    EPAD = _cdiv(E, 32768) * 32768

    src = edge_index[0]
    dst = edge_index[1]
    pad = EPAD - E
    srcp = jnp.concatenate([src, jnp.zeros((pad,), src.dtype)])
    dstp = jnp.concatenate([dst, jnp.full((pad,), N, dst.dtype)])
    src2d = srcp.reshape(EPAD // 128, 128)
    dst2d = dstp.reshape(EPAD // 128, 128)

    degp = _make_deg_kernel(NPAD, EPAD)(dst2d)
    d, hs1 = _tc_pre(NPAD, N, Dhid)(degp, x, W1)
    acc1 = _make_sc_edge_kernel(NPAD, EPAD, Dhid // 2)(hs1, src2d, dst2d)
    (hs2,) = _tc_mid(NPAD, N, Dhid, Dout)(acc1, d, b1, W2)
    acc2 = _make_sc_edge_kernel(NPAD, EPAD, Dout // 2)(hs2, src2d, dst2d)
    out = _tc_post(NPAD, N, Dout)(acc2, d, b2)
    return out


# speedup vs baseline: 26.7714x; 2.0251x over previous
import functools

import jax
import jax.numpy as jnp
from jax import lax
from jax.experimental import pallas as pl
from jax.experimental.pallas import tpu as pltpu
from jax.experimental.pallas import tpu_sc as plsc

NC = 2
NS = 16
LN = 16


def _cdiv(a, b):
    return (a + b - 1) // b


def _sc_mesh():
    return plsc.VectorSubcoreMesh(
        core_axis_name="c", subcore_axis_name="s", num_cores=NC, num_subcores=NS
    )


def _make_deg_kernel(NPAD, EPAD):
    EPT = EPAD // (NC * NS)
    CH = EPT // 128
    SLAB = NPAD // NS

    @functools.partial(
        pl.kernel,
        out_type=jax.ShapeDtypeStruct((NC, NPAD), jnp.float32),
        mesh=_sc_mesh(),
        scratch_types=[
            pltpu.VMEM((CH, 128), jnp.int32),
            pltpu.VMEM((128,), jnp.float32),
            pltpu.VMEM((SLAB,), jnp.float32),
            pltpu.VMEM_SHARED((NPAD,), jnp.float32),
        ],
    )
    def deg_kernel(dst_hbm, out_hbm, idx_v, ones_v, zbuf, sdeg):
        c = lax.axis_index("c")
        s = lax.axis_index("s")
        wid = s * NC + c

        def zero(i, _):
            zbuf[pl.ds(i * LN, LN)] = jnp.zeros((LN,), jnp.float32)
            return 0

        lax.fori_loop(0, SLAB // LN, zero, 0)

        def one(i, _):
            ones_v[pl.ds(i * LN, LN)] = jnp.full((LN,), 1.0, jnp.float32)
            return 0

        lax.fori_loop(0, 128 // LN, one, 0)
        pltpu.sync_copy(zbuf, sdeg.at[pl.ds(s * SLAB, SLAB)])
        pltpu.sync_copy(dst_hbm.at[pl.ds(wid * CH, CH)], idx_v)
        plsc.subcore_barrier()

        def body(j, _):
            pltpu.sync_copy(ones_v, sdeg.at[idx_v.at[j]], add=True)
            return 0

        lax.fori_loop(0, CH, body, 0)
        plsc.subcore_barrier()
        pltpu.sync_copy(sdeg.at[pl.ds(s * SLAB, SLAB)], out_hbm.at[c, pl.ds(s * SLAB, SLAB)])

    return deg_kernel


def _make_sc_edge_kernel(NPAD, EPAD, D):
    EPT = EPAD // NS
    CH = EPT // 128
    SC_CH = 16
    NSUP = CH // SC_CH
    SLAB = NPAD // NS

    @functools.partial(
        pl.kernel,
        out_type=jax.ShapeDtypeStruct((NC, NPAD, D), jnp.float32),
        mesh=_sc_mesh(),
        scratch_types=[
            pltpu.VMEM((SC_CH, 128), jnp.int32),
            pltpu.VMEM((SC_CH, 128), jnp.int32),
            pltpu.VMEM((128, D), jnp.float32),
            pltpu.VMEM((128, D), jnp.float32),
            pltpu.VMEM_SHARED((NPAD, D), jnp.float32),
            pltpu.VMEM_SHARED((NPAD, D), jnp.float32),
            pltpu.SemaphoreType.DMA,
            pltpu.SemaphoreType.DMA,
        ],
        compiler_params=pltpu.CompilerParams(use_tc_tiling_on_sc=False),
    )
    def edge_kernel(h_hbm, src_hbm, dst_hbm, out_hbm,
                    sidx, didx, ga, gb, accum, table, sa, sb):
        c = lax.axis_index("c")
        s = lax.axis_index("s")
        r0 = s * SLAB
        pltpu.sync_copy(h_hbm.at[c, pl.ds(r0, SLAB)], accum.at[pl.ds(r0, SLAB)])
        pltpu.sync_copy(h_hbm.at[c, pl.ds(r0, SLAB)], table.at[pl.ds(r0, SLAB)])
        plsc.subcore_barrier()

        def body(g, _):
            base = s * CH + g * SC_CH
            pltpu.sync_copy(src_hbm.at[pl.ds(base, SC_CH)], sidx)
            pltpu.sync_copy(dst_hbm.at[pl.ds(base, SC_CH)], didx)
            pltpu.async_copy(table.at[sidx.at[0]], ga, sa)
            for j in range(SC_CH):
                buf, sem = (ga, sa) if j % 2 == 0 else (gb, sb)
                nbuf, nsem = (gb, sb) if j % 2 == 0 else (ga, sa)
                pltpu.make_async_copy(table.at[sidx.at[j]], buf, sem).wait()
                if j + 1 < SC_CH:
                    pltpu.async_copy(table.at[sidx.at[j + 1]], nbuf, nsem)
                pltpu.sync_copy(buf, accum.at[didx.at[j]], add=True)
            return 0

        lax.fori_loop(0, NSUP, body, 0)
        plsc.subcore_barrier()
        pltpu.sync_copy(accum.at[pl.ds(r0, SLAB)], out_hbm.at[c, pl.ds(r0, SLAB)])

    return edge_kernel


def _tc_pre(NPAD, N, Dhid):
    H = Dhid // 2

    def body(dp_ref, x_ref, w_ref, d_ref, hs_ref):
        deg = jnp.sum(dp_ref[...], axis=0) + 1.0
        d = lax.rsqrt(deg)
        dcol = d.reshape(NPAD, 1)
        d_ref[...] = dcol
        h = jnp.dot(x_ref[...], w_ref[...], preferred_element_type=jnp.float32)
        hs = h * dcol[:N]
        hs_ref[0, :N, :] = hs[:, :H]
        hs_ref[1, :N, :] = hs[:, H:]

    return pl.pallas_call(
        body,
        out_shape=[
            jax.ShapeDtypeStruct((NPAD, 1), jnp.float32),
            jax.ShapeDtypeStruct((NC, NPAD, H), jnp.float32),
        ],
    )


def _tc_mid(NPAD, N, Dhid, Dout):
    H = Dout // 2

    def body(a_ref, d_ref, b1_ref, w2_ref, hs_ref):
        acc = jnp.concatenate([a_ref[0, :N, :], a_ref[1, :N, :]], axis=1)
        dcol = d_ref[:N]
        h = jnp.maximum(acc * dcol + b1_ref[...], 0.0)
        h2 = jnp.dot(h, w2_ref[...], preferred_element_type=jnp.float32)
        hs = h2 * dcol
        hs_ref[0, :N, :] = hs[:, :H]
        hs_ref[1, :N, :] = hs[:, H:]

    return pl.pallas_call(
        body,
        out_shape=[jax.ShapeDtypeStruct((NC, NPAD, H), jnp.float32)],
    )


def _tc_post(NPAD, N, Dout):
    def body(a_ref, d_ref, b2_ref, out_ref):
        acc = jnp.concatenate([a_ref[0, :N, :], a_ref[1, :N, :]], axis=1)
        out_ref[...] = acc * d_ref[:N] + b2_ref[...]

    return pl.pallas_call(
        body,
        out_shape=jax.ShapeDtypeStruct((N, Dout), jnp.float32),
    )


def kernel(x, edge_index, W1, b1, W2, b2):
    N, Din = x.shape
    E = edge_index.shape[1]
    Dhid = W1.shape[1]
    Dout = W2.shape[1]

    NPAD = _cdiv(N + 1, 256) * 256
    EPAD = _cdiv(E, 32768) * 32768

    src = edge_index[0]
    dst = edge_index[1]
    pad = EPAD - E
    srcp = jnp.concatenate([src, jnp.zeros((pad,), src.dtype)])
    dstp = jnp.concatenate([dst, jnp.full((pad,), N, dst.dtype)])
    src2d = srcp.reshape(EPAD // 128, 128)
    dst2d = dstp.reshape(EPAD // 128, 128)

    degp = _make_deg_kernel(NPAD, EPAD)(dst2d)
    d, hs1 = _tc_pre(NPAD, N, Dhid)(degp, x, W1)
    acc1 = _make_sc_edge_kernel(NPAD, EPAD, Dhid // 2)(hs1, src2d, dst2d)
    (hs2,) = _tc_mid(NPAD, N, Dhid, Dout)(acc1, d, b1, W2)
    acc2 = _make_sc_edge_kernel(NPAD, EPAD, Dout // 2)(hs2, src2d, dst2d)
    out = _tc_post(NPAD, N, Dout)(acc2, d, b2)
    return out


# speedup vs baseline: 28.2233x; 1.0542x over previous
import functools

import jax
import jax.numpy as jnp
from jax import lax
from jax.experimental import pallas as pl
from jax.experimental.pallas import tpu as pltpu
from jax.experimental.pallas import tpu_sc as plsc

NC = 2
NS = 16
LN = 16


def _cdiv(a, b):
    return (a + b - 1) // b


def _sc_mesh():
    return plsc.VectorSubcoreMesh(
        core_axis_name="c", subcore_axis_name="s", num_cores=NC, num_subcores=NS
    )


def _make_deg_kernel(NPAD, EPAD):
    EPT = EPAD // (NC * NS)
    CH = EPT // 128
    SLAB = NPAD // NS

    @functools.partial(
        pl.kernel,
        out_type=jax.ShapeDtypeStruct((NC, NPAD), jnp.float32),
        mesh=_sc_mesh(),
        scratch_types=[
            pltpu.VMEM((CH, 128), jnp.int32),
            pltpu.VMEM((128,), jnp.float32),
            pltpu.VMEM((SLAB,), jnp.float32),
            pltpu.VMEM_SHARED((NPAD,), jnp.float32),
        ],
    )
    def deg_kernel(dst_hbm, out_hbm, idx_v, ones_v, zbuf, sdeg):
        c = lax.axis_index("c")
        s = lax.axis_index("s")
        wid = s * NC + c

        def zero(i, _):
            zbuf[pl.ds(i * LN, LN)] = jnp.zeros((LN,), jnp.float32)
            return 0

        lax.fori_loop(0, SLAB // LN, zero, 0)

        def one(i, _):
            ones_v[pl.ds(i * LN, LN)] = jnp.full((LN,), 1.0, jnp.float32)
            return 0

        lax.fori_loop(0, 128 // LN, one, 0)
        pltpu.sync_copy(zbuf, sdeg.at[pl.ds(s * SLAB, SLAB)])
        pltpu.sync_copy(dst_hbm.at[pl.ds(wid * CH, CH)], idx_v)
        plsc.subcore_barrier()

        def body(j, _):
            pltpu.sync_copy(ones_v, sdeg.at[idx_v.at[j]], add=True)
            return 0

        lax.fori_loop(0, CH, body, 0)
        plsc.subcore_barrier()
        pltpu.sync_copy(sdeg.at[pl.ds(s * SLAB, SLAB)], out_hbm.at[c, pl.ds(s * SLAB, SLAB)])

    return deg_kernel


def _make_sc_edge_kernel(NPAD, EPAD, D):
    EPT = EPAD // NS
    CH = EPT // 128
    SC_CH = 16
    NSUP = CH // SC_CH
    SLAB = NPAD // NS

    @functools.partial(
        pl.kernel,
        out_type=jax.ShapeDtypeStruct((NC, NPAD, D), jnp.float32),
        mesh=_sc_mesh(),
        scratch_types=[
            pltpu.VMEM((2, SC_CH, 128), jnp.int32),
            pltpu.VMEM((2, SC_CH, 128), jnp.int32),
            pltpu.VMEM((128, D), jnp.float32),
            pltpu.VMEM((128, D), jnp.float32),
            pltpu.VMEM_SHARED((NPAD, D), jnp.float32),
            pltpu.VMEM_SHARED((NPAD, D), jnp.float32),
            pltpu.SemaphoreType.DMA,
            pltpu.SemaphoreType.DMA,
            pltpu.SemaphoreType.DMA,
            pltpu.SemaphoreType.DMA,
        ],
        compiler_params=pltpu.CompilerParams(use_tc_tiling_on_sc=False),
    )
    def edge_kernel(h_hbm, src_hbm, dst_hbm, out_hbm,
                    sidx, didx, ga, gb, accum, table, sa, sb, si0, si1):
        c = lax.axis_index("c")
        s = lax.axis_index("s")
        r0 = s * SLAB
        sisems = (si0, si1)

        def idx_start(g, p):
            base = s * CH + g * SC_CH
            pltpu.make_async_copy(
                src_hbm.at[pl.ds(base, SC_CH)], sidx.at[p], sisems[p]).start()
            pltpu.make_async_copy(
                dst_hbm.at[pl.ds(base, SC_CH)], didx.at[p], sisems[p]).start()

        def idx_wait(g, p):
            base = s * CH + g * SC_CH
            pltpu.make_async_copy(
                src_hbm.at[pl.ds(base, SC_CH)], sidx.at[p], sisems[p]).wait()
            pltpu.make_async_copy(
                dst_hbm.at[pl.ds(base, SC_CH)], didx.at[p], sisems[p]).wait()

        idx_start(0, 0)
        pltpu.sync_copy(h_hbm.at[c, pl.ds(r0, SLAB)], accum.at[pl.ds(r0, SLAB)])
        pltpu.sync_copy(h_hbm.at[c, pl.ds(r0, SLAB)], table.at[pl.ds(r0, SLAB)])
        plsc.subcore_barrier()

        def super_chunk(g, p):
            idx_wait(g, p)

            @pl.when(g + 1 < NSUP)
            def _():
                idx_start(g + 1, 1 - p)

            pltpu.async_copy(table.at[sidx.at[p, 0]], ga, sa)
            for j in range(SC_CH):
                buf, sem = (ga, sa) if j % 2 == 0 else (gb, sb)
                nbuf, nsem = (gb, sb) if j % 2 == 0 else (ga, sa)
                pltpu.make_async_copy(table.at[sidx.at[p, j]], buf, sem).wait()
                if j + 1 < SC_CH:
                    pltpu.async_copy(table.at[sidx.at[p, j + 1]], nbuf, nsem)
                pltpu.sync_copy(buf, accum.at[didx.at[p, j]], add=True)

        def body(t, _):
            super_chunk(2 * t, 0)
            super_chunk(2 * t + 1, 1)
            return 0

        lax.fori_loop(0, NSUP // 2, body, 0)
        if NSUP % 2:
            super_chunk(NSUP - 1, (NSUP - 1) % 2)
        plsc.subcore_barrier()
        pltpu.sync_copy(accum.at[pl.ds(r0, SLAB)], out_hbm.at[c, pl.ds(r0, SLAB)])

    return edge_kernel


def _tc_mm1(N, Din, Dhid):
    def body(x_ref, w_ref, h_ref):
        h_ref[...] = jnp.dot(x_ref[...], w_ref[...],
                             preferred_element_type=jnp.float32)

    return pl.pallas_call(
        body, out_shape=jax.ShapeDtypeStruct((N, Dhid), jnp.float32))


def _tc_pre(NPAD, N, Dhid):
    H = Dhid // 2

    def body(dp_ref, h_ref, d_ref, hs_ref):
        deg = jnp.sum(dp_ref[...], axis=0) + 1.0
        d = lax.rsqrt(deg)
        dcol = d.reshape(NPAD, 1)
        d_ref[...] = dcol
        hs = h_ref[...] * dcol[:N]
        hs_ref[0, :N, :] = hs[:, :H]
        hs_ref[1, :N, :] = hs[:, H:]

    return pl.pallas_call(
        body,
        out_shape=[
            jax.ShapeDtypeStruct((NPAD, 1), jnp.float32),
            jax.ShapeDtypeStruct((NC, NPAD, H), jnp.float32),
        ],
    )


def _tc_mid(NPAD, N, Dhid, Dout):
    H = Dout // 2

    def body(a_ref, d_ref, b1_ref, w2_ref, hs_ref):
        acc = jnp.concatenate([a_ref[0, :N, :], a_ref[1, :N, :]], axis=1)
        dcol = d_ref[:N]
        h = jnp.maximum(acc * dcol + b1_ref[...], 0.0)
        h2 = jnp.dot(h, w2_ref[...], preferred_element_type=jnp.float32)
        hs = h2 * dcol
        hs_ref[0, :N, :] = hs[:, :H]
        hs_ref[1, :N, :] = hs[:, H:]

    return pl.pallas_call(
        body,
        out_shape=[jax.ShapeDtypeStruct((NC, NPAD, H), jnp.float32)],
    )


def _tc_post(NPAD, N, Dout):
    def body(a_ref, d_ref, b2_ref, out_ref):
        acc = jnp.concatenate([a_ref[0, :N, :], a_ref[1, :N, :]], axis=1)
        out_ref[...] = acc * d_ref[:N] + b2_ref[...]

    return pl.pallas_call(
        body,
        out_shape=jax.ShapeDtypeStruct((N, Dout), jnp.float32),
    )


def kernel(x, edge_index, W1, b1, W2, b2):
    N, Din = x.shape
    E = edge_index.shape[1]
    Dhid = W1.shape[1]
    Dout = W2.shape[1]

    NPAD = _cdiv(N + 1, 256) * 256
    EPAD = _cdiv(E, 32768) * 32768

    src = edge_index[0]
    dst = edge_index[1]
    pad = EPAD - E
    srcp = jnp.concatenate([src, jnp.zeros((pad,), src.dtype)])
    dstp = jnp.concatenate([dst, jnp.full((pad,), N, dst.dtype)])
    src2d = srcp.reshape(EPAD // 128, 128)
    dst2d = dstp.reshape(EPAD // 128, 128)

    degp = _make_deg_kernel(NPAD, EPAD)(dst2d)
    h1 = _tc_mm1(N, Din, Dhid)(x, W1)
    d, hs1 = _tc_pre(NPAD, N, Dhid)(degp, h1)
    acc1 = _make_sc_edge_kernel(NPAD, EPAD, Dhid // 2)(hs1, src2d, dst2d)
    (hs2,) = _tc_mid(NPAD, N, Dhid, Dout)(acc1, d, b1, W2)
    acc2 = _make_sc_edge_kernel(NPAD, EPAD, Dout // 2)(hs2, src2d, dst2d)
    out = _tc_post(NPAD, N, Dout)(acc2, d, b2)
    return out


# speedup vs baseline: 29.3994x; 1.0417x over previous
import functools

import jax
import jax.numpy as jnp
from jax import lax
from jax.experimental import pallas as pl
from jax.experimental.pallas import tpu as pltpu
from jax.experimental.pallas import tpu_sc as plsc

NC = 2
NS = 16
LN = 16


def _cdiv(a, b):
    return (a + b - 1) // b


def _sc_mesh():
    return plsc.VectorSubcoreMesh(
        core_axis_name="c", subcore_axis_name="s", num_cores=NC, num_subcores=NS
    )


def _make_deg_kernel(NPAD, EPAD):
    EPT = EPAD // (NC * NS)
    CH = EPT // 128
    SLAB = NPAD // NS

    @functools.partial(
        pl.kernel,
        out_type=jax.ShapeDtypeStruct((NC, NPAD), jnp.float32),
        mesh=_sc_mesh(),
        scratch_types=[
            pltpu.VMEM((CH, 128), jnp.int32),
            pltpu.VMEM((128,), jnp.float32),
            pltpu.VMEM((SLAB,), jnp.float32),
            pltpu.VMEM_SHARED((NPAD,), jnp.float32),
        ],
    )
    def deg_kernel(dst_hbm, out_hbm, idx_v, ones_v, zbuf, sdeg):
        c = lax.axis_index("c")
        s = lax.axis_index("s")
        wid = s * NC + c

        def zero(i, _):
            zbuf[pl.ds(i * LN, LN)] = jnp.zeros((LN,), jnp.float32)
            return 0

        lax.fori_loop(0, SLAB // LN, zero, 0)

        def one(i, _):
            ones_v[pl.ds(i * LN, LN)] = jnp.full((LN,), 1.0, jnp.float32)
            return 0

        lax.fori_loop(0, 128 // LN, one, 0)
        pltpu.sync_copy(zbuf, sdeg.at[pl.ds(s * SLAB, SLAB)])
        pltpu.sync_copy(dst_hbm.at[pl.ds(wid * CH, CH)], idx_v)
        plsc.subcore_barrier()

        def body(j, _):
            pltpu.sync_copy(ones_v, sdeg.at[idx_v.at[j]], add=True)
            return 0

        lax.fori_loop(0, CH, body, 0)
        plsc.subcore_barrier()
        pltpu.sync_copy(sdeg.at[pl.ds(s * SLAB, SLAB)], out_hbm.at[c, pl.ds(s * SLAB, SLAB)])

    return deg_kernel


def _make_sc_edge_kernel(NPAD, EPAD, D, final=None):
    EPT = EPAD // NS
    CH = EPT // 128
    SC_CH = 16
    NSUP = CH // SC_CH
    SLAB = NPAD // NS

    if final is None:
        out_type = jax.ShapeDtypeStruct((NC, NPAD, D), jnp.float32)
        extra_scratch = []
    else:
        N, DoutF = final
        out_type = jax.ShapeDtypeStruct((N, DoutF), jnp.float32)
        extra_scratch = [
            pltpu.VMEM((SLAB,), jnp.float32),
            pltpu.VMEM((DoutF,), jnp.float32),
        ]

    @functools.partial(
        pl.kernel,
        out_type=out_type,
        mesh=_sc_mesh(),
        scratch_types=extra_scratch + [
            pltpu.VMEM((2, SC_CH, 128), jnp.int32),
            pltpu.VMEM((2, SC_CH, 128), jnp.int32),
            pltpu.VMEM((128, D), jnp.float32),
            pltpu.VMEM((128, D), jnp.float32),
            pltpu.VMEM_SHARED((NPAD, D), jnp.float32),
            pltpu.VMEM_SHARED((NPAD, D), jnp.float32),
            pltpu.SemaphoreType.DMA,
            pltpu.SemaphoreType.DMA,
            pltpu.SemaphoreType.DMA,
            pltpu.SemaphoreType.DMA,
        ],
        compiler_params=pltpu.CompilerParams(use_tc_tiling_on_sc=False),
    )
    def edge_kernel(*args):
        if final is None:
            (h_hbm, src_hbm, dst_hbm, out_hbm,
             sidx, didx, ga, gb, accum, table, sa, sb, si0, si1) = args
            d_hbm = b_hbm = dv = bv = None
        else:
            (h_hbm, src_hbm, dst_hbm, d_hbm, b_hbm, out_hbm, dv, bv,
             sidx, didx, ga, gb, accum, table, sa, sb, si0, si1) = args
        c = lax.axis_index("c")
        s = lax.axis_index("s")
        r0 = s * SLAB
        sisems = (si0, si1)

        def idx_start(g, p):
            base = s * CH + g * SC_CH
            pltpu.make_async_copy(
                src_hbm.at[pl.ds(base, SC_CH)], sidx.at[p], sisems[p]).start()
            pltpu.make_async_copy(
                dst_hbm.at[pl.ds(base, SC_CH)], didx.at[p], sisems[p]).start()

        def idx_wait(g, p):
            base = s * CH + g * SC_CH
            pltpu.make_async_copy(
                src_hbm.at[pl.ds(base, SC_CH)], sidx.at[p], sisems[p]).wait()
            pltpu.make_async_copy(
                dst_hbm.at[pl.ds(base, SC_CH)], didx.at[p], sisems[p]).wait()

        idx_start(0, 0)
        pltpu.sync_copy(h_hbm.at[c, pl.ds(r0, SLAB)], accum.at[pl.ds(r0, SLAB)])
        pltpu.sync_copy(h_hbm.at[c, pl.ds(r0, SLAB)], table.at[pl.ds(r0, SLAB)])
        plsc.subcore_barrier()

        def super_chunk(g, p):
            idx_wait(g, p)

            @pl.when(g + 1 < NSUP)
            def _():
                idx_start(g + 1, 1 - p)

            pltpu.async_copy(table.at[sidx.at[p, 0]], ga, sa)
            for j in range(SC_CH):
                buf, sem = (ga, sa) if j % 2 == 0 else (gb, sb)
                nbuf, nsem = (gb, sb) if j % 2 == 0 else (ga, sa)
                pltpu.make_async_copy(table.at[sidx.at[p, j]], buf, sem).wait()
                if j + 1 < SC_CH:
                    pltpu.async_copy(table.at[sidx.at[p, j + 1]], nbuf, nsem)
                pltpu.sync_copy(buf, accum.at[didx.at[p, j]], add=True)

        def body(t, _):
            super_chunk(2 * t, 0)
            super_chunk(2 * t + 1, 1)
            return 0

        lax.fori_loop(0, NSUP // 2, body, 0)
        if NSUP % 2:
            super_chunk(NSUP - 1, (NSUP - 1) % 2)
        plsc.subcore_barrier()
        if final is None:
            pltpu.sync_copy(accum.at[pl.ds(r0, SLAB)],
                            out_hbm.at[c, pl.ds(r0, SLAB)])
        else:
            pltpu.sync_copy(d_hbm.at[pl.ds(r0, SLAB)], dv)
            pltpu.sync_copy(b_hbm, bv)
            col0 = c * D
            bregs = [bv[pl.ds(col0 + 16 * k2, 16)] for k2 in range(D // 16)]

            def scale_chunk(row0, local0, nrows):
                pltpu.sync_copy(accum.at[pl.ds(row0, nrows)],
                                ga.at[pl.ds(0, nrows)])

                def rowgroup(g2, _):
                    base = g2 * LN
                    dvec = dv[pl.ds(local0 + base, LN)]
                    for r in range(LN):
                        dval = dvec[r]
                        for k2 in range(D // 16):
                            seg = ga[base + r, pl.ds(16 * k2, 16)]
                            ga[base + r, pl.ds(16 * k2, 16)] = (
                                seg * dval + bregs[k2])
                    return 0

                lax.fori_loop(0, nrows // LN, rowgroup, 0)
                pltpu.sync_copy(
                    ga.at[pl.ds(0, nrows)],
                    out_hbm.at[pl.ds(row0, nrows), pl.ds(col0, D)])

            full_tiles = N // SLAB
            rem = N % SLAB

            @pl.when(s < full_tiles)
            def _():
                for k in range(SLAB // 128):
                    scale_chunk(r0 + k * 128, k * 128, 128)

            if rem:
                @pl.when(s == full_tiles)
                def _():
                    nfull = rem // 128
                    for k in range(nfull):
                        scale_chunk(r0 + k * 128, k * 128, 128)
                    tail = rem - nfull * 128
                    if tail:
                        scale_chunk(r0 + nfull * 128, nfull * 128, tail)

    return edge_kernel


def _tc_mm1(N, Din, Dhid):
    def body(x_ref, w_ref, h_ref):
        h_ref[...] = jnp.dot(x_ref[...], w_ref[...],
                             preferred_element_type=jnp.float32)

    return pl.pallas_call(
        body, out_shape=jax.ShapeDtypeStruct((N, Dhid), jnp.float32))


def _tc_pre(NPAD, N, Dhid):
    H = Dhid // 2

    def body(dp_ref, h_ref, d_ref, hs_ref):
        deg = jnp.sum(dp_ref[...], axis=0) + 1.0
        d = lax.rsqrt(deg)
        dcol = d.reshape(NPAD, 1)
        d_ref[...] = dcol
        hs = h_ref[...] * dcol[:N]
        hs_ref[0, :N, :] = hs[:, :H]
        hs_ref[1, :N, :] = hs[:, H:]

    return pl.pallas_call(
        body,
        out_shape=[
            jax.ShapeDtypeStruct((NPAD, 1), jnp.float32),
            jax.ShapeDtypeStruct((NC, NPAD, H), jnp.float32),
        ],
    )


def _tc_mid(NPAD, N, Dhid, Dout):
    H = Dout // 2

    def body(a_ref, d_ref, b1_ref, w2_ref, hs_ref):
        acc = jnp.concatenate([a_ref[0, :N, :], a_ref[1, :N, :]], axis=1)
        dcol = d_ref[:N]
        h = jnp.maximum(acc * dcol + b1_ref[...], 0.0)
        h2 = jnp.dot(h, w2_ref[...], preferred_element_type=jnp.float32)
        hs = h2 * dcol
        hs_ref[0, :N, :] = hs[:, :H]
        hs_ref[1, :N, :] = hs[:, H:]

    return pl.pallas_call(
        body,
        out_shape=[jax.ShapeDtypeStruct((NC, NPAD, H), jnp.float32)],
    )


def _tc_post(NPAD, N, Dout):
    def body(a_ref, d_ref, b2_ref, out_ref):
        acc = jnp.concatenate([a_ref[0, :N, :], a_ref[1, :N, :]], axis=1)
        out_ref[...] = acc * d_ref[:N] + b2_ref[...]

    return pl.pallas_call(
        body,
        out_shape=jax.ShapeDtypeStruct((N, Dout), jnp.float32),
    )


def kernel(x, edge_index, W1, b1, W2, b2):
    N, Din = x.shape
    E = edge_index.shape[1]
    Dhid = W1.shape[1]
    Dout = W2.shape[1]

    NPAD = _cdiv(N + 1, 256) * 256
    EPAD = _cdiv(E, 32768) * 32768

    src = edge_index[0]
    dst = edge_index[1]
    pad = EPAD - E
    srcp = jnp.concatenate([src, jnp.zeros((pad,), src.dtype)])
    dstp = jnp.concatenate([dst, jnp.full((pad,), N, dst.dtype)])
    src2d = srcp.reshape(EPAD // 128, 128)
    dst2d = dstp.reshape(EPAD // 128, 128)

    degp = _make_deg_kernel(NPAD, EPAD)(dst2d)
    h1 = _tc_mm1(N, Din, Dhid)(x, W1)
    d, hs1 = _tc_pre(NPAD, N, Dhid)(degp, h1)
    acc1 = _make_sc_edge_kernel(NPAD, EPAD, Dhid // 2)(hs1, src2d, dst2d)
    (hs2,) = _tc_mid(NPAD, N, Dhid, Dout)(acc1, d, b1, W2)
    out = _make_sc_edge_kernel(NPAD, EPAD, Dout // 2, final=(N, Dout))(
        hs2, src2d, dst2d, d.reshape(NPAD), b2)
    return out


# speedup vs baseline: 30.0709x; 1.0228x over previous
import functools

import jax
import jax.numpy as jnp
from jax import lax
from jax.experimental import pallas as pl
from jax.experimental.pallas import tpu as pltpu
from jax.experimental.pallas import tpu_sc as plsc

NC = 2
NS = 16
LN = 16


def _cdiv(a, b):
    return (a + b - 1) // b


def _sc_mesh():
    return plsc.VectorSubcoreMesh(
        core_axis_name="c", subcore_axis_name="s", num_cores=NC, num_subcores=NS
    )


def _make_deg_kernel(NPAD, EPAD):
    EPT = EPAD // (NC * NS)
    CH = EPT // 128
    SLAB = NPAD // NS

    @functools.partial(
        pl.kernel,
        out_type=jax.ShapeDtypeStruct((NC, NPAD), jnp.float32),
        mesh=_sc_mesh(),
        scratch_types=[
            pltpu.VMEM((CH, 128), jnp.int32),
            pltpu.VMEM((128,), jnp.float32),
            pltpu.VMEM((SLAB,), jnp.float32),
            pltpu.VMEM_SHARED((NPAD,), jnp.float32),
        ],
    )
    def deg_kernel(dst_hbm, out_hbm, idx_v, ones_v, zbuf, sdeg):
        c = lax.axis_index("c")
        s = lax.axis_index("s")
        wid = s * NC + c

        def zero(i, _):
            zbuf[pl.ds(i * LN, LN)] = jnp.zeros((LN,), jnp.float32)
            return 0

        lax.fori_loop(0, SLAB // LN, zero, 0)

        def one(i, _):
            ones_v[pl.ds(i * LN, LN)] = jnp.full((LN,), 1.0, jnp.float32)
            return 0

        lax.fori_loop(0, 128 // LN, one, 0)
        pltpu.sync_copy(zbuf, sdeg.at[pl.ds(s * SLAB, SLAB)])
        pltpu.sync_copy(dst_hbm.at[pl.ds(wid * CH, CH)], idx_v)
        plsc.subcore_barrier()

        def body(j, _):
            pltpu.sync_copy(ones_v, sdeg.at[idx_v.at[j]], add=True)
            return 0

        lax.fori_loop(0, CH, body, 0)
        plsc.subcore_barrier()
        pltpu.sync_copy(sdeg.at[pl.ds(s * SLAB, SLAB)], out_hbm.at[c, pl.ds(s * SLAB, SLAB)])

    return deg_kernel


def _make_sc_edge_kernel(NPAD, EPAD, D, final=None, pre=None):
    EPT = EPAD // NS
    CH = EPT // 128
    SC_CH = 16
    NSUP = CH // SC_CH
    SLAB = NPAD // NS

    if final is None:
        out_type = [jax.ShapeDtypeStruct((NC, NPAD, D), jnp.float32)]
        extra_scratch = []
    else:
        N, DoutF = final
        out_type = [jax.ShapeDtypeStruct((N, DoutF), jnp.float32)]
        extra_scratch = [
            pltpu.VMEM((SLAB,), jnp.float32),
            pltpu.VMEM((DoutF,), jnp.float32),
        ]
    if pre is not None:
        out_type = out_type + [jax.ShapeDtypeStruct((NC, NPAD), jnp.float32)]
        extra_scratch = extra_scratch + [
            pltpu.VMEM((SLAB,), jnp.float32),
            pltpu.VMEM((2, SLAB), jnp.float32),
        ]

    @functools.partial(
        pl.kernel,
        out_type=out_type,
        mesh=_sc_mesh(),
        scratch_types=extra_scratch + [
            pltpu.VMEM((2, SC_CH, 128), jnp.int32),
            pltpu.VMEM((2, SC_CH, 128), jnp.int32),
            pltpu.VMEM((128, D), jnp.float32),
            pltpu.VMEM((128, D), jnp.float32),
            pltpu.VMEM_SHARED((NPAD, D), jnp.float32),
            pltpu.VMEM_SHARED((NPAD, D), jnp.float32),
            pltpu.SemaphoreType.DMA,
            pltpu.SemaphoreType.DMA,
            pltpu.SemaphoreType.DMA,
            pltpu.SemaphoreType.DMA,
        ],
        compiler_params=pltpu.CompilerParams(use_tc_tiling_on_sc=False),
    )
    def edge_kernel(*args):
        if pre is not None:
            (h_hbm, degp_hbm, src_hbm, dst_hbm, out_hbm, dout_hbm,
             dvp, dpb, sidx, didx, ga, gb, accum, table, sa, sb, si0, si1) = args
        elif final is None:
            (h_hbm, src_hbm, dst_hbm, out_hbm,
             sidx, didx, ga, gb, accum, table, sa, sb, si0, si1) = args
            d_hbm = b_hbm = dv = bv = None
        else:
            (h_hbm, src_hbm, dst_hbm, d_hbm, b_hbm, out_hbm, dv, bv,
             sidx, didx, ga, gb, accum, table, sa, sb, si0, si1) = args
        c = lax.axis_index("c")
        s = lax.axis_index("s")
        r0 = s * SLAB
        sisems = (si0, si1)

        def idx_start(g, p):
            base = s * CH + g * SC_CH
            pltpu.make_async_copy(
                src_hbm.at[pl.ds(base, SC_CH)], sidx.at[p], sisems[p]).start()
            pltpu.make_async_copy(
                dst_hbm.at[pl.ds(base, SC_CH)], didx.at[p], sisems[p]).start()

        def idx_wait(g, p):
            base = s * CH + g * SC_CH
            pltpu.make_async_copy(
                src_hbm.at[pl.ds(base, SC_CH)], sidx.at[p], sisems[p]).wait()
            pltpu.make_async_copy(
                dst_hbm.at[pl.ds(base, SC_CH)], didx.at[p], sisems[p]).wait()

        idx_start(0, 0)
        if pre is None:
            pltpu.sync_copy(h_hbm.at[c, pl.ds(r0, SLAB)],
                            accum.at[pl.ds(r0, SLAB)])
            pltpu.sync_copy(h_hbm.at[c, pl.ds(r0, SLAB)],
                            table.at[pl.ds(r0, SLAB)])
        else:
            Np, _Dfull = pre
            pltpu.sync_copy(degp_hbm.at[0, pl.ds(r0, SLAB)], dpb.at[0])
            pltpu.sync_copy(degp_hbm.at[1, pl.ds(r0, SLAB)], dpb.at[1])

            def dgrp(g2, _):
                b = g2 * LN
                xv = dpb[0, pl.ds(b, LN)] + dpb[1, pl.ds(b, LN)] + 1.0
                iv = lax.bitcast_convert_type(xv, jnp.int32)
                y = lax.bitcast_convert_type(
                    jnp.int32(0x5F3759DF) - (iv >> 1), jnp.float32)
                for _i in range(3):
                    y = y * (1.5 - 0.5 * xv * y * y)
                dvp[pl.ds(b, LN)] = y
                return 0

            lax.fori_loop(0, SLAB // LN, dgrp, 0)
            pltpu.sync_copy(dvp, dout_hbm.at[c, pl.ds(r0, SLAB)])

            def stage_chunk(row0, local0, nrows):
                pltpu.sync_copy(
                    h_hbm.at[pl.ds(row0, nrows), pl.ds(c * D, D)],
                    gb.at[pl.ds(0, nrows)])

                def rg(g2, _):
                    b = g2 * LN
                    dvec = dvp[pl.ds(local0 + b, LN)]
                    for r in range(LN):
                        dval = dvec[r]
                        for k2 in range(D // 16):
                            gb[b + r, pl.ds(16 * k2, 16)] = (
                                gb[b + r, pl.ds(16 * k2, 16)] * dval)
                    return 0

                lax.fori_loop(0, nrows // LN, rg, 0)
                pltpu.sync_copy(gb.at[pl.ds(0, nrows)],
                                table.at[pl.ds(row0, nrows)])
                pltpu.sync_copy(gb.at[pl.ds(0, nrows)],
                                accum.at[pl.ds(row0, nrows)])

            pfull = Np // SLAB
            prem = Np % SLAB

            @pl.when(s < pfull)
            def _():
                for k in range(SLAB // 128):
                    stage_chunk(r0 + k * 128, k * 128, 128)

            if prem:
                @pl.when(s == pfull)
                def _():
                    nf = prem // 128
                    for k in range(nf):
                        stage_chunk(r0 + k * 128, k * 128, 128)
                    ptail = prem - nf * 128
                    if ptail:
                        stage_chunk(r0 + nf * 128, nf * 128, ptail)
        plsc.subcore_barrier()

        def super_chunk(g, p):
            idx_wait(g, p)

            @pl.when(g + 1 < NSUP)
            def _():
                idx_start(g + 1, 1 - p)

            pltpu.async_copy(table.at[sidx.at[p, 0]], ga, sa)
            for j in range(SC_CH):
                buf, sem = (ga, sa) if j % 2 == 0 else (gb, sb)
                nbuf, nsem = (gb, sb) if j % 2 == 0 else (ga, sa)
                pltpu.make_async_copy(table.at[sidx.at[p, j]], buf, sem).wait()
                if j + 1 < SC_CH:
                    pltpu.async_copy(table.at[sidx.at[p, j + 1]], nbuf, nsem)
                pltpu.sync_copy(buf, accum.at[didx.at[p, j]], add=True)

        def body(t, _):
            super_chunk(2 * t, 0)
            super_chunk(2 * t + 1, 1)
            return 0

        lax.fori_loop(0, NSUP // 2, body, 0)
        if NSUP % 2:
            super_chunk(NSUP - 1, (NSUP - 1) % 2)
        plsc.subcore_barrier()
        if final is None:
            pltpu.sync_copy(accum.at[pl.ds(r0, SLAB)],
                            out_hbm.at[c, pl.ds(r0, SLAB)])
        else:
            pltpu.sync_copy(d_hbm.at[c, pl.ds(r0, SLAB)], dv)
            pltpu.sync_copy(b_hbm, bv)
            col0 = c * D
            bregs = [bv[pl.ds(col0 + 16 * k2, 16)] for k2 in range(D // 16)]

            def scale_chunk(row0, local0, nrows):
                pltpu.sync_copy(accum.at[pl.ds(row0, nrows)],
                                ga.at[pl.ds(0, nrows)])

                def rowgroup(g2, _):
                    base = g2 * LN
                    dvec = dv[pl.ds(local0 + base, LN)]
                    for r in range(LN):
                        dval = dvec[r]
                        for k2 in range(D // 16):
                            seg = ga[base + r, pl.ds(16 * k2, 16)]
                            ga[base + r, pl.ds(16 * k2, 16)] = (
                                seg * dval + bregs[k2])
                    return 0

                lax.fori_loop(0, nrows // LN, rowgroup, 0)
                pltpu.sync_copy(
                    ga.at[pl.ds(0, nrows)],
                    out_hbm.at[pl.ds(row0, nrows), pl.ds(col0, D)])

            full_tiles = N // SLAB
            rem = N % SLAB

            @pl.when(s < full_tiles)
            def _():
                for k in range(SLAB // 128):
                    scale_chunk(r0 + k * 128, k * 128, 128)

            if rem:
                @pl.when(s == full_tiles)
                def _():
                    nfull = rem // 128
                    for k in range(nfull):
                        scale_chunk(r0 + k * 128, k * 128, 128)
                    tail = rem - nfull * 128
                    if tail:
                        scale_chunk(r0 + nfull * 128, nfull * 128, tail)

    return edge_kernel


def _tc_mm1(N, Din, Dhid):
    def body(x_ref, w_ref, h_ref):
        h_ref[...] = jnp.dot(x_ref[...], w_ref[...],
                             preferred_element_type=jnp.float32)

    return pl.pallas_call(
        body, out_shape=jax.ShapeDtypeStruct((N, Dhid), jnp.float32))


def _tc_pre(NPAD, N, Dhid):
    H = Dhid // 2

    def body(dp_ref, h_ref, d_ref, hs_ref):
        deg = jnp.sum(dp_ref[...], axis=0) + 1.0
        d = lax.rsqrt(deg)
        dcol = d.reshape(NPAD, 1)
        d_ref[...] = dcol
        hs = h_ref[...] * dcol[:N]
        hs_ref[0, :N, :] = hs[:, :H]
        hs_ref[1, :N, :] = hs[:, H:]

    return pl.pallas_call(
        body,
        out_shape=[
            jax.ShapeDtypeStruct((NPAD, 1), jnp.float32),
            jax.ShapeDtypeStruct((NC, NPAD, H), jnp.float32),
        ],
    )


def _tc_mid(NPAD, N, Dhid, Dout):
    H = Dout // 2

    def body(a_ref, d_ref, b1_ref, w2_ref, hs_ref):
        acc = jnp.concatenate([a_ref[0, :N, :], a_ref[1, :N, :]], axis=1)
        dcol = d_ref[0].reshape(NPAD, 1)[:N]
        h = jnp.maximum(acc * dcol + b1_ref[...], 0.0)
        h2 = jnp.dot(h, w2_ref[...], preferred_element_type=jnp.float32)
        hs = h2 * dcol
        hs_ref[0, :N, :] = hs[:, :H]
        hs_ref[1, :N, :] = hs[:, H:]

    return pl.pallas_call(
        body,
        out_shape=[jax.ShapeDtypeStruct((NC, NPAD, H), jnp.float32)],
    )


def _tc_post(NPAD, N, Dout):
    def body(a_ref, d_ref, b2_ref, out_ref):
        acc = jnp.concatenate([a_ref[0, :N, :], a_ref[1, :N, :]], axis=1)
        out_ref[...] = acc * d_ref[:N] + b2_ref[...]

    return pl.pallas_call(
        body,
        out_shape=jax.ShapeDtypeStruct((N, Dout), jnp.float32),
    )


def kernel(x, edge_index, W1, b1, W2, b2):
    N, Din = x.shape
    E = edge_index.shape[1]
    Dhid = W1.shape[1]
    Dout = W2.shape[1]

    NPAD = _cdiv(N + 1, 256) * 256
    EPAD = _cdiv(E, 32768) * 32768

    src = edge_index[0]
    dst = edge_index[1]
    pad = EPAD - E
    srcp = jnp.concatenate([src, jnp.zeros((pad,), src.dtype)])
    dstp = jnp.concatenate([dst, jnp.full((pad,), N, dst.dtype)])
    src2d = srcp.reshape(EPAD // 128, 128)
    dst2d = dstp.reshape(EPAD // 128, 128)

    degp = _make_deg_kernel(NPAD, EPAD)(dst2d)
    h1 = _tc_mm1(N, Din, Dhid)(x, W1)
    acc1, dout = _make_sc_edge_kernel(NPAD, EPAD, Dhid // 2, pre=(N, Dhid))(
        h1, degp, src2d, dst2d)
    (hs2,) = _tc_mid(NPAD, N, Dhid, Dout)(acc1, dout, b1, W2)
    (out,) = _make_sc_edge_kernel(NPAD, EPAD, Dout // 2, final=(N, Dout))(
        hs2, src2d, dst2d, dout, b2)
    return out
